# baseline (speedup 1.0000x reference)
import functools

import jax
import jax.numpy as jnp
from jax import lax
from jax.experimental import pallas as pl
from jax.experimental.pallas import tpu as pltpu
from jax.experimental.pallas import tpu_sc as plsc

N = 8192
BR = 256
N_BLOCKS = N // BR

NC = 2
NS = 16
NW = NC * NS
PER_W = N // NW
L = 16


def _tc_argmax_rowsum(f_ref, idx_ref, err_ref):
    i = pl.program_id(0)
    f = f_ref[...]
    rowsum = jnp.sum(f, axis=1, keepdims=True)
    m = jnp.max(f, axis=1, keepdims=True)
    cols = lax.broadcasted_iota(jnp.int32, (BR, N), 1)
    amax = jnp.min(jnp.where(f == m, cols, N), axis=1, keepdims=True)
    idx_ref[...] = amax
    err = jnp.sum(jnp.abs(1.0 - rowsum))

    @pl.when(i == 0)
    def _init():
        err_ref[0, 0] = err

    @pl.when(i != 0)
    def _acc():
        err_ref[0, 0] += err


_tc_pass = pl.pallas_call(
    _tc_argmax_rowsum,
    grid=(N_BLOCKS,),
    in_specs=[pl.BlockSpec((BR, N), lambda i: (i, 0))],
    out_specs=[
        pl.BlockSpec((BR, 1), lambda i: (i, 0)),
        pl.BlockSpec(memory_space=pltpu.SMEM),
    ],
    out_shape=[
        jax.ShapeDtypeStruct((N, 1), jnp.int32),
        jax.ShapeDtypeStruct((1, 1), jnp.float32),
    ],
)


def _sc_gather_body(dist_hbm, col_hbm, out_hbm, col_v, vals_v, acc_v, sem):
    wid = lax.axis_index("s") * NC + lax.axis_index("c")
    base = wid * PER_W
    pltpu.sync_copy(col_hbm.at[pl.ds(base, PER_W)], col_v)

    def chunk(k, acc):
        v16 = col_v[pl.ds(k * L, L)]
        for j in range(L):
            c = v16[j]
            cb = pl.multiple_of((c >> 7) << 7, 128)
            rt = pl.multiple_of(base + k * L + (j & ~7), 8)
            pltpu.async_copy(
                dist_hbm.at[pl.ds(rt, 8), pl.ds(cb, 128)], vals_v.at[j], sem
            )
        for j in range(L):
            pltpu.make_async_copy(
                dist_hbm.at[pl.ds(0, 8), pl.ds(0, 128)], vals_v.at[j], sem
            ).wait()
        iota16 = lax.iota(jnp.int32, L)
        for j in range(L):
            lane = jnp.full((L,), v16[j] & 127, jnp.int32)
            for s in range(128 // L):
                row = vals_v[j, j & 7, pl.ds(s * L, L)]
                acc = acc + jnp.where(iota16 + s * L == lane, row, 0.0)
        return acc

    acc = lax.fori_loop(0, PER_W // L, chunk, jnp.zeros((L,), jnp.float32))
    acc_v[...] = acc
    pltpu.sync_copy(acc_v, out_hbm.at[wid])


@functools.lru_cache(maxsize=1)
def _sc_gather():
    return functools.partial(
        pl.kernel,
        mesh=plsc.VectorSubcoreMesh(core_axis_name="c", subcore_axis_name="s"),
        out_type=jax.ShapeDtypeStruct((NW, L), jnp.float32),
        scratch_types=[
            pltpu.VMEM((PER_W,), jnp.int32),
            pltpu.VMEM((L, 8, 128), jnp.float32),
            pltpu.VMEM((L,), jnp.float32),
            pltpu.SemaphoreType.DMA,
        ],
    )(_sc_gather_body)


def kernel(feature, distances, target):
    del target
    col_idx, err2 = _tc_pass(feature)
    partials = _sc_gather()(distances, col_idx.reshape(-1))
    return err2[0, 0] + jnp.sum(partials)


# speedup vs baseline: 1.0257x; 1.0257x over previous
import functools

import jax
import jax.numpy as jnp
from jax import lax
from jax.experimental import pallas as pl
from jax.experimental.pallas import tpu as pltpu
from jax.experimental.pallas import tpu_sc as plsc

N = 8192
BR = 256
N_BLOCKS = N // BR

NC = 2
NS = 16
NW = NC * NS
PER_W = N // NW
L = 16


def _tc_argmax_rowsum(f_ref, idx_ref, err_ref):
    i = pl.program_id(0)
    f = f_ref[...]
    rowsum = jnp.sum(f, axis=1, keepdims=True)
    m = jnp.max(f, axis=1, keepdims=True)
    cols = lax.broadcasted_iota(jnp.int32, (BR, N), 1)
    amax = jnp.min(jnp.where(f == m, cols, N), axis=1, keepdims=True)
    idx_ref[...] = amax
    err = jnp.sum(jnp.abs(1.0 - rowsum))

    @pl.when(i == 0)
    def _init():
        err_ref[0, 0] = err

    @pl.when(i != 0)
    def _acc():
        err_ref[0, 0] += err


_tc_pass = pl.pallas_call(
    _tc_argmax_rowsum,
    grid=(N_BLOCKS,),
    in_specs=[pl.BlockSpec((BR, N), lambda i: (i, 0))],
    out_specs=[
        pl.BlockSpec((BR, 1), lambda i: (i, 0)),
        pl.BlockSpec(memory_space=pltpu.SMEM),
    ],
    out_shape=[
        jax.ShapeDtypeStruct((N, 1), jnp.int32),
        jax.ShapeDtypeStruct((1, 1), jnp.float32),
    ],
)


NBUF = 4
NCHUNK = PER_W // L
EPOCHS = NCHUNK // NBUF


def _sc_gather_body(dist_hbm, col_hbm, out_hbm, col_v, vals_v, acc_v, *sems):
    wid = lax.axis_index("s") * NC + lax.axis_index("c")
    base = wid * PER_W
    pltpu.sync_copy(col_hbm.at[pl.ds(base, PER_W)], col_v)
    iota16 = lax.iota(jnp.int32, L)

    def fire(k, b):
        v16 = col_v[pl.ds(k * L, L)]
        for j in range(L):
            c = v16[j]
            cb = pl.multiple_of((c >> 7) << 7, 128)
            rt = pl.multiple_of(base + k * L + (j & ~7), 8)
            pltpu.async_copy(
                dist_hbm.at[pl.ds(rt, 8), pl.ds(cb, 128)],
                vals_v.at[b * L + j], sems[b]
            )

    for b in range(NBUF):
        fire(b, b)

    def epoch(e, acc):
        for b in range(NBUF):
            k = e * NBUF + b
            for j in range(L):
                pltpu.make_async_copy(
                    dist_hbm.at[pl.ds(0, 8), pl.ds(0, 128)],
                    vals_v.at[b * L + j], sems[b]
                ).wait()
            v16 = col_v[pl.ds(k * L, L)]
            for j in range(L):
                lane = jnp.full((L,), v16[j] & 127, jnp.int32)
                for s in range(128 // L):
                    row = vals_v[b * L + j, j & 7, pl.ds(s * L, L)]
                    acc = acc + jnp.where(iota16 + s * L == lane, row, 0.0)

            @pl.when(e + 1 < EPOCHS)
            def _refill():
                fire(k + NBUF, b)
        return acc

    acc = lax.fori_loop(0, EPOCHS, epoch, jnp.zeros((L,), jnp.float32))
    acc_v[...] = acc
    pltpu.sync_copy(acc_v, out_hbm.at[wid])


@functools.lru_cache(maxsize=1)
def _sc_gather():
    return functools.partial(
        pl.kernel,
        mesh=plsc.VectorSubcoreMesh(core_axis_name="c", subcore_axis_name="s"),
        out_type=jax.ShapeDtypeStruct((NW, L), jnp.float32),
        scratch_types=[
            pltpu.VMEM((PER_W,), jnp.int32),
            pltpu.VMEM((NBUF * L, 8, 128), jnp.float32),
            pltpu.VMEM((L,), jnp.float32),
        ] + [pltpu.SemaphoreType.DMA] * NBUF,
    )(_sc_gather_body)


def kernel(feature, distances, target):
    del target
    col_idx, err2 = _tc_pass(feature)
    partials = _sc_gather()(distances, col_idx.reshape(-1))
    return err2[0, 0] + jnp.sum(partials)
